# Initial kernel scaffold; baseline (speedup 1.0000x reference)
import jax
import jax.numpy as jnp
from jax.experimental import pallas as pl


def kernel(x, edge_index, W0, b0, W1, b1, W2, b2):
    raise NotImplementedError("write your pallas kernel here")


# speedup vs baseline: 5.4704x; 5.4704x over previous
import functools

import jax
import jax.numpy as jnp
from jax import lax
from jax.experimental import pallas as pl
from jax.experimental.pallas import tpu as pltpu
from jax.experimental.pallas import tpu_sc as plsc

N = 10000
D = 128
E = 320000
NC = 2
NS = 16
NW = NC * NS
NPAD = 10240
NHALF = NPAD // NC
CHUNK = 128
KCH = 160
EPAD = NS * CHUNK * KCH
RPT = NHALF // NS
BR = 1024
IGN = -1


def _remap(sidx, didx, off):
    def row(i, _):
        def col(k, _):
            sl = pl.ds(k * 16, 16)
            d = didx[i, sl] - off
            ok = (d >= 0) & (d < NHALF)
            didx[i, sl] = jnp.where(ok, d, IGN)
            sidx[i, sl] = jnp.where(ok, sidx[i, sl], IGN)
            return 0
        return lax.fori_loop(0, CHUNK // 16, col, 0)
    lax.fori_loop(0, KCH, row, 0)


def _zero_acc_stripe(rows, acc, sid):
    z = jnp.zeros((16,), jnp.float32)
    def row(i, _):
        def col(k, _):
            rows[i, pl.ds(k * 16, 16)] = z
            return 0
        return lax.fori_loop(0, D // 16, col, 0)
    lax.fori_loop(0, CHUNK, row, 0)
    for off, n in _stripe_pieces():
        pltpu.sync_copy(rows.at[pl.ds(0, n)],
                        acc.at[pl.ds(sid * RPT + off, n)])


def _stripe_pieces():
    pieces, off = [], 0
    while off < RPT:
        n = min(CHUNK, RPT - off)
        pieces.append((off, n))
        off += n
    return pieces


def _writeback(acc, stage, out_hbm, cid, sid):
    for off, n in _stripe_pieces():
        pltpu.sync_copy(acc.at[pl.ds(sid * RPT + off, n)], stage.at[pl.ds(0, n)])
        pltpu.sync_copy(stage.at[pl.ds(0, n)],
                        out_hbm.at[pl.ds(cid * NHALF + sid * RPT + off, n)])


def _sc_deg_body(dst_hbm, out_hbm, didx, ones_v, acc, sem):
    cid = lax.axis_index("c")
    sid = lax.axis_index("s")

    _zero_acc_stripe(ones_v, acc, sid)
    one = jnp.ones((16,), jnp.float32)
    def fill(i, _):
        def col(k, _):
            ones_v[i, pl.ds(k * 16, 16)] = one
            return 0
        return lax.fori_loop(0, D // 16, col, 0)
    lax.fori_loop(0, CHUNK, fill, 0)
    pltpu.sync_copy(dst_hbm.at[pl.ds(sid * KCH, KCH)], didx)
    _remap(didx, didx, cid * NHALF)
    plsc.subcore_barrier()

    def chunk(j, _):
        pltpu.sync_copy(ones_v,
                        acc.at[plsc.Indices(didx.at[j], ignored_value=IGN)],
                        add=True)
        return 0

    lax.fori_loop(0, KCH, chunk, 0)
    plsc.subcore_barrier()
    _writeback(acc, ones_v, out_hbm, cid, sid)


@functools.lru_cache(maxsize=None)
def _sc_deg():
    return pl.kernel(
        _sc_deg_body,
        out_type=jax.ShapeDtypeStruct((NPAD, D), jnp.float32),
        mesh=plsc.VectorSubcoreMesh(core_axis_name="c", subcore_axis_name="s"),
        scratch_types=[
            pltpu.VMEM((KCH, CHUNK), jnp.int32),
            pltpu.VMEM((CHUNK, D), jnp.float32),
            pltpu.VMEM_SHARED((NHALF, D), jnp.float32),
            pltpu.SemaphoreType.DMA,
        ],
    )


def _sc_scatter_body(g_hbm, src_hbm, dst_hbm, out_hbm, sidx, didx, rows, acc, gsem):
    cid = lax.axis_index("c")
    sid = lax.axis_index("s")

    _zero_acc_stripe(rows, acc, sid)
    pltpu.sync_copy(src_hbm.at[pl.ds(sid * KCH, KCH)], sidx)
    pltpu.sync_copy(dst_hbm.at[pl.ds(sid * KCH, KCH)], didx)
    _remap(sidx, didx, cid * NHALF)
    plsc.subcore_barrier()

    def chunk(j, _):
        pltpu.async_copy(
            g_hbm.at[plsc.Indices(sidx.at[j], ignored_value=IGN)], rows, gsem
        ).wait()
        pltpu.sync_copy(rows,
                        acc.at[plsc.Indices(didx.at[j], ignored_value=IGN)],
                        add=True)
        return 0

    lax.fori_loop(0, KCH, chunk, 0)
    plsc.subcore_barrier()
    _writeback(acc, rows, out_hbm, cid, sid)


@functools.lru_cache(maxsize=None)
def _sc_scatter():
    return pl.kernel(
        _sc_scatter_body,
        out_type=jax.ShapeDtypeStruct((NPAD, D), jnp.float32),
        mesh=plsc.VectorSubcoreMesh(core_axis_name="c", subcore_axis_name="s"),
        scratch_types=[
            pltpu.VMEM((KCH, CHUNK), jnp.int32),
            pltpu.VMEM((KCH, CHUNK), jnp.int32),
            pltpu.VMEM((CHUNK, D), jnp.float32),
            pltpu.VMEM_SHARED((NHALF, D), jnp.float32),
            pltpu.SemaphoreType.DMA,
        ],
    )


def _tc_first_body(x_ref, w_ref, d_ref, o_ref):
    dis = lax.rsqrt(d_ref[...] + 1.0)
    o_ref[...] = dis * jnp.dot(x_ref[...], w_ref[...],
                               preferred_element_type=jnp.float32)


def _tc_mid_body(s_ref, g_ref, d_ref, b_ref, w_ref, o_ref):
    dis = lax.rsqrt(d_ref[...] + 1.0)
    y = jnp.maximum(dis * (s_ref[...] + g_ref[...]) + b_ref[...], 0.0)
    o_ref[...] = dis * jnp.dot(y, w_ref[...], preferred_element_type=jnp.float32)


def _tc_final_body(s_ref, g_ref, d_ref, b_ref, o_ref):
    dis = lax.rsqrt(d_ref[...] + 1.0)
    o_ref[...] = jnp.maximum(dis * (s_ref[...] + g_ref[...]) + b_ref[...], 0.0)


_row_spec = pl.BlockSpec((BR, D), lambda i: (i, 0))
_w_spec = pl.BlockSpec((D, D), lambda i: (0, 0))
_b_spec = pl.BlockSpec((1, D), lambda i: (0, 0))
_out_sds = jax.ShapeDtypeStruct((NPAD, D), jnp.float32)
_grid = (NPAD // BR,)


def _tc_first(x, W0, deg):
    return pl.pallas_call(
        _tc_first_body, grid=_grid,
        in_specs=[_row_spec, _w_spec, _row_spec],
        out_specs=_row_spec, out_shape=_out_sds)(x, W0, deg)


def _tc_mid(S, g, deg, b, W):
    return pl.pallas_call(
        _tc_mid_body, grid=_grid,
        in_specs=[_row_spec, _row_spec, _row_spec, _b_spec, _w_spec],
        out_specs=_row_spec, out_shape=_out_sds)(S, g, deg, b, W)


def _tc_final(S, g, deg, b):
    return pl.pallas_call(
        _tc_final_body, grid=_grid,
        in_specs=[_row_spec, _row_spec, _row_spec, _b_spec],
        out_specs=_row_spec, out_shape=_out_sds)(S, g, deg, b)


def kernel(x, edge_index, W0, b0, W1, b1, W2, b2):
    src = edge_index[0].astype(jnp.int32)
    dst = edge_index[1].astype(jnp.int32)
    padv = jnp.full((EPAD - E,), N, jnp.int32)
    srcp = jnp.concatenate([src, padv]).reshape(NS * KCH, CHUNK)
    dstp = jnp.concatenate([dst, padv]).reshape(NS * KCH, CHUNK)
    x_pad = jnp.pad(x, ((0, NPAD - N), (0, 0)))

    deg = _sc_deg()(dstp)
    g0 = _tc_first(x_pad, W0, deg)
    S0 = _sc_scatter()(g0, srcp, dstp)
    g1 = _tc_mid(S0, g0, deg, b0.reshape(1, D), W1)
    S1 = _sc_scatter()(g1, srcp, dstp)
    g2 = _tc_mid(S1, g1, deg, b1.reshape(1, D), W2)
    S2 = _sc_scatter()(g2, srcp, dstp)
    y = _tc_final(S2, g2, deg, b2.reshape(1, D))
    return y[:N]


# speedup vs baseline: 6.0742x; 1.1104x over previous
import functools

import jax
import jax.numpy as jnp
from jax import lax
from jax.experimental import pallas as pl
from jax.experimental.pallas import tpu as pltpu
from jax.experimental.pallas import tpu_sc as plsc

N = 10000
D = 128
E = 320000
NC = 2
NS = 16
NW = NC * NS
NPAD = 10240
NHALF = NPAD // NC
CHUNK = 128
KCH = 160
EPAD = NS * CHUNK * KCH
RPT = NHALF // NS
BR = 1024
NB = 4
NBS = 2
IGN = -1


def _remap(sidx, didx, off):
    def row(i, _):
        def col(k, _):
            sl = pl.ds(k * 16, 16)
            d = didx[i, sl] - off
            ok = (d >= 0) & (d < NHALF)
            didx[i, sl] = jnp.where(ok, d, IGN)
            sidx[i, sl] = jnp.where(ok, sidx[i, sl], IGN)
            return 0
        return lax.fori_loop(0, CHUNK // 16, col, 0)
    lax.fori_loop(0, KCH, row, 0)


def _zero_acc_stripe(rows, acc, sid):
    z = jnp.zeros((16,), jnp.float32)
    def row(i, _):
        def col(k, _):
            rows[i, pl.ds(k * 16, 16)] = z
            return 0
        return lax.fori_loop(0, D // 16, col, 0)
    lax.fori_loop(0, CHUNK, row, 0)
    for off, n in _stripe_pieces():
        pltpu.sync_copy(rows.at[pl.ds(0, n)],
                        acc.at[pl.ds(sid * RPT + off, n)])


def _stripe_pieces():
    pieces, off = [], 0
    while off < RPT:
        n = min(CHUNK, RPT - off)
        pieces.append((off, n))
        off += n
    return pieces


def _writeback(acc, stage, out_hbm, cid, sid):
    for off, n in _stripe_pieces():
        pltpu.sync_copy(acc.at[pl.ds(sid * RPT + off, n)], stage.at[pl.ds(0, n)])
        pltpu.sync_copy(stage.at[pl.ds(0, n)],
                        out_hbm.at[pl.ds(cid * NHALF + sid * RPT + off, n)])


def _sc_deg_body(dst_hbm, out_hbm, didx, ones_v, acc, s0, s1, s2, s3):
    sems = (s0, s1, s2, s3)
    cid = lax.axis_index("c")
    sid = lax.axis_index("s")

    _zero_acc_stripe(ones_v, acc, sid)
    one = jnp.ones((16,), jnp.float32)
    def fill(i, _):
        def col(k, _):
            ones_v[i, pl.ds(k * 16, 16)] = one
            return 0
        return lax.fori_loop(0, D // 16, col, 0)
    lax.fori_loop(0, CHUNK, fill, 0)
    pltpu.sync_copy(dst_hbm.at[pl.ds(sid * KCH, KCH)], didx)
    _remap(didx, didx, cid * NHALF)
    plsc.subcore_barrier()

    def group(jj, _):
        cps = [pltpu.async_copy(
                   ones_v,
                   acc.at[plsc.Indices(didx.at[jj * NB + b], ignored_value=IGN)],
                   sems[b], add=True)
               for b in range(NB)]
        for cp in cps:
            cp.wait()
        return 0

    lax.fori_loop(0, KCH // NB, group, 0)
    plsc.subcore_barrier()
    _writeback(acc, ones_v, out_hbm, cid, sid)


@functools.lru_cache(maxsize=None)
def _sc_deg():
    return pl.kernel(
        _sc_deg_body,
        out_type=jax.ShapeDtypeStruct((NPAD, D), jnp.float32),
        mesh=plsc.VectorSubcoreMesh(core_axis_name="c", subcore_axis_name="s"),
        scratch_types=[
            pltpu.VMEM((KCH, CHUNK), jnp.int32),
            pltpu.VMEM((CHUNK, D), jnp.float32),
            pltpu.VMEM_SHARED((NHALF, D), jnp.float32),
        ] + [pltpu.SemaphoreType.DMA] * NB,
    )


def _sc_scatter_body(g_hbm, src_hbm, dst_hbm, out_hbm, sidx, didx, rows,
                     acc, g0, g1, t0, t1):
    gsems = (g0, g1)
    ssems = (t0, t1)
    cid = lax.axis_index("c")
    sid = lax.axis_index("s")

    _zero_acc_stripe(rows.at[0], acc, sid)
    pltpu.sync_copy(src_hbm.at[pl.ds(sid * KCH, KCH)], sidx)
    pltpu.sync_copy(dst_hbm.at[pl.ds(sid * KCH, KCH)], didx)
    _remap(sidx, didx, cid * NHALF)
    plsc.subcore_barrier()

    def gather(j, b):
        return pltpu.async_copy(
            g_hbm.at[plsc.Indices(sidx.at[j], ignored_value=IGN)],
            rows.at[b], gsems[b])

    def scatter(j, b):
        return pltpu.async_copy(
            rows.at[b],
            acc.at[plsc.Indices(didx.at[j], ignored_value=IGN)],
            ssems[b], add=True)

    def group(jj, _):
        cps = [gather(jj * NBS + b, b) for b in range(NBS)]
        scs = []
        for b in range(NBS):
            cps[b].wait()
            scs.append(scatter(jj * NBS + b, b))
        for sc in scs:
            sc.wait()
        return 0

    lax.fori_loop(0, KCH // NBS, group, 0)

    plsc.subcore_barrier()
    _writeback(acc, rows.at[0], out_hbm, cid, sid)


@functools.lru_cache(maxsize=None)
def _sc_scatter():
    return pl.kernel(
        _sc_scatter_body,
        out_type=jax.ShapeDtypeStruct((NPAD, D), jnp.float32),
        mesh=plsc.VectorSubcoreMesh(core_axis_name="c", subcore_axis_name="s"),
        scratch_types=[
            pltpu.VMEM((KCH, CHUNK), jnp.int32),
            pltpu.VMEM((KCH, CHUNK), jnp.int32),
            pltpu.VMEM((NBS, CHUNK, D), jnp.float32),
            pltpu.VMEM_SHARED((NHALF, D), jnp.float32),
        ] + [pltpu.SemaphoreType.DMA] * (2 * NBS),
    )


def _tc_first_body(x_ref, w_ref, d_ref, o_ref):
    dis = lax.rsqrt(d_ref[...] + 1.0)
    o_ref[...] = dis * jnp.dot(x_ref[...], w_ref[...],
                               preferred_element_type=jnp.float32)


def _tc_mid_body(s_ref, g_ref, d_ref, b_ref, w_ref, o_ref):
    dis = lax.rsqrt(d_ref[...] + 1.0)
    y = jnp.maximum(dis * (s_ref[...] + g_ref[...]) + b_ref[...], 0.0)
    o_ref[...] = dis * jnp.dot(y, w_ref[...], preferred_element_type=jnp.float32)


def _tc_final_body(s_ref, g_ref, d_ref, b_ref, o_ref):
    dis = lax.rsqrt(d_ref[...] + 1.0)
    o_ref[...] = jnp.maximum(dis * (s_ref[...] + g_ref[...]) + b_ref[...], 0.0)


_row_spec = pl.BlockSpec((BR, D), lambda i: (i, 0))
_w_spec = pl.BlockSpec((D, D), lambda i: (0, 0))
_b_spec = pl.BlockSpec((1, D), lambda i: (0, 0))
_out_sds = jax.ShapeDtypeStruct((NPAD, D), jnp.float32)
_grid = (NPAD // BR,)


def _tc_first(x, W0, deg):
    return pl.pallas_call(
        _tc_first_body, grid=_grid,
        in_specs=[_row_spec, _w_spec, _row_spec],
        out_specs=_row_spec, out_shape=_out_sds)(x, W0, deg)


def _tc_mid(S, g, deg, b, W):
    return pl.pallas_call(
        _tc_mid_body, grid=_grid,
        in_specs=[_row_spec, _row_spec, _row_spec, _b_spec, _w_spec],
        out_specs=_row_spec, out_shape=_out_sds)(S, g, deg, b, W)


def _tc_final(S, g, deg, b):
    return pl.pallas_call(
        _tc_final_body, grid=_grid,
        in_specs=[_row_spec, _row_spec, _row_spec, _b_spec],
        out_specs=_row_spec, out_shape=_out_sds)(S, g, deg, b)


def kernel(x, edge_index, W0, b0, W1, b1, W2, b2):
    src = edge_index[0].astype(jnp.int32)
    dst = edge_index[1].astype(jnp.int32)
    padv = jnp.full((EPAD - E,), N, jnp.int32)
    srcp = jnp.concatenate([src, padv]).reshape(NS * KCH, CHUNK)
    dstp = jnp.concatenate([dst, padv]).reshape(NS * KCH, CHUNK)
    x_pad = jnp.pad(x, ((0, NPAD - N), (0, 0)))

    deg = _sc_deg()(dstp)
    g0 = _tc_first(x_pad, W0, deg)
    S0 = _sc_scatter()(g0, srcp, dstp)
    g1 = _tc_mid(S0, g0, deg, b0.reshape(1, D), W1)
    S1 = _sc_scatter()(g1, srcp, dstp)
    g2 = _tc_mid(S1, g1, deg, b1.reshape(1, D), W2)
    S2 = _sc_scatter()(g2, srcp, dstp)
    y = _tc_final(S2, g2, deg, b2.reshape(1, D))
    return y[:N]


# speedup vs baseline: 14.6275x; 2.4081x over previous
import functools

import jax
import jax.numpy as jnp
from jax import lax
from jax.experimental import pallas as pl
from jax.experimental.pallas import tpu as pltpu
from jax.experimental.pallas import tpu_sc as plsc

N = 10000
D = 128
E = 320000
NC = 2
NS = 16
NW = NC * NS
NPAD = 10240
NHALF = NPAD // NC
CHUNK = 128
KCH = 160
EPAD = NS * CHUNK * KCH
RPT = NHALF // NS
BR = 1024
NB = 4
NBS = 2
IGN = -1


def _remap(sidx, didx, off):
    def row(i, _):
        def col(k, _):
            sl = pl.ds(k * 16, 16)
            d = didx[i, sl] - off
            ok = (d >= 0) & (d < NHALF)
            didx[i, sl] = jnp.where(ok, d, IGN)
            sidx[i, sl] = jnp.where(ok, sidx[i, sl], IGN)
            return 0
        return lax.fori_loop(0, CHUNK // 16, col, 0)
    lax.fori_loop(0, KCH, row, 0)


def _zero_acc_stripe(rows, acc, sid):
    z = jnp.zeros((16,), jnp.float32)
    def row(i, _):
        def col(k, _):
            rows[i, pl.ds(k * 16, 16)] = z
            return 0
        return lax.fori_loop(0, D // 16, col, 0)
    lax.fori_loop(0, CHUNK, row, 0)
    for off, n in _stripe_pieces():
        pltpu.sync_copy(rows.at[pl.ds(0, n)],
                        acc.at[pl.ds(sid * RPT + off, n)])


def _stripe_pieces():
    pieces, off = [], 0
    while off < RPT:
        n = min(CHUNK, RPT - off)
        pieces.append((off, n))
        off += n
    return pieces


def _writeback(acc, stage, out_hbm, cid, sid):
    for off, n in _stripe_pieces():
        pltpu.sync_copy(acc.at[pl.ds(sid * RPT + off, n)], stage.at[pl.ds(0, n)])
        pltpu.sync_copy(stage.at[pl.ds(0, n)],
                        out_hbm.at[pl.ds(cid * NHALF + sid * RPT + off, n)])


def _sc_deg_body(dst_hbm, out_hbm, didx, ones_v, acc, s0, s1, s2, s3):
    sems = (s0, s1, s2, s3)
    cid = lax.axis_index("c")
    sid = lax.axis_index("s")

    _zero_acc_stripe(ones_v, acc, sid)
    one = jnp.ones((16,), jnp.float32)
    def fill(i, _):
        def col(k, _):
            ones_v[i, pl.ds(k * 16, 16)] = one
            return 0
        return lax.fori_loop(0, D // 16, col, 0)
    lax.fori_loop(0, CHUNK, fill, 0)
    pltpu.sync_copy(dst_hbm.at[pl.ds(sid * KCH, KCH)], didx)
    _remap(didx, didx, cid * NHALF)
    plsc.subcore_barrier()

    def group(jj, _):
        cps = [pltpu.async_copy(
                   ones_v,
                   acc.at[plsc.Indices(didx.at[jj * NB + b], ignored_value=IGN)],
                   sems[b], add=True)
               for b in range(NB)]
        for cp in cps:
            cp.wait()
        return 0

    lax.fori_loop(0, KCH // NB, group, 0)
    plsc.subcore_barrier()
    _writeback(acc, ones_v, out_hbm, cid, sid)


@functools.lru_cache(maxsize=None)
def _sc_deg():
    return pl.kernel(
        _sc_deg_body,
        out_type=jax.ShapeDtypeStruct((NPAD, D), jnp.float32),
        mesh=plsc.VectorSubcoreMesh(core_axis_name="c", subcore_axis_name="s"),
        scratch_types=[
            pltpu.VMEM((KCH, CHUNK), jnp.int32),
            pltpu.VMEM((CHUNK, D), jnp.float32),
            pltpu.VMEM_SHARED((NHALF, D), jnp.float32),
        ] + [pltpu.SemaphoreType.DMA] * NB,
    )


def _sc_scatter_body(g_hbm, src_hbm, dst_hbm, out_hbm, sidx, didx, rows,
                     acc, g0, g1, t0, t1):
    gsems = (g0, g1)
    ssems = (t0, t1)
    cid = lax.axis_index("c")
    sid = lax.axis_index("s")

    _zero_acc_stripe(rows.at[0], acc, sid)
    pltpu.sync_copy(src_hbm.at[pl.ds(sid * KCH, KCH)], sidx)
    pltpu.sync_copy(dst_hbm.at[pl.ds(sid * KCH, KCH)], didx)
    _remap(sidx, didx, cid * NHALF)
    plsc.subcore_barrier()

    def gather(j, b):
        return pltpu.async_copy(
            g_hbm.at[plsc.Indices(sidx.at[j], ignored_value=IGN)],
            rows.at[b], gsems[b])

    def scatter(j, b):
        return pltpu.async_copy(
            rows.at[b],
            acc.at[plsc.Indices(didx.at[j], ignored_value=IGN)],
            ssems[b], add=True)

    def group(jj, _):
        cps = [gather(jj * NBS + b, b) for b in range(NBS)]
        scs = []
        for b in range(NBS):
            cps[b].wait()
            scs.append(scatter(jj * NBS + b, b))
        for sc in scs:
            sc.wait()
        return 0

    lax.fori_loop(0, KCH // NBS, group, 0)

    plsc.subcore_barrier()
    _writeback(acc, rows.at[0], out_hbm, cid, sid)


@functools.lru_cache(maxsize=None)
def _sc_scatter():
    return pl.kernel(
        _sc_scatter_body,
        out_type=jax.ShapeDtypeStruct((NPAD, D), jnp.float32),
        mesh=plsc.VectorSubcoreMesh(core_axis_name="c", subcore_axis_name="s"),
        scratch_types=[
            pltpu.VMEM((KCH, CHUNK), jnp.int32),
            pltpu.VMEM((KCH, CHUNK), jnp.int32),
            pltpu.VMEM((NBS, CHUNK, D), jnp.float32),
            pltpu.VMEM_SHARED((NHALF, D), jnp.float32),
        ] + [pltpu.SemaphoreType.DMA] * (2 * NBS),
    )


def _tc_first_body(x_ref, w_ref, d_ref, o_ref):
    dis = lax.rsqrt(d_ref[...] + 1.0)
    o_ref[...] = dis * jnp.dot(x_ref[...], w_ref[...],
                               preferred_element_type=jnp.float32)


def _tc_mid_body(s_ref, g_ref, d_ref, b_ref, w_ref, o_ref):
    dis = lax.rsqrt(d_ref[...] + 1.0)
    y = jnp.maximum(dis * (s_ref[...] + g_ref[...]) + b_ref[...], 0.0)
    o_ref[...] = dis * jnp.dot(y, w_ref[...], preferred_element_type=jnp.float32)


def _tc_final_body(s_ref, g_ref, d_ref, b_ref, o_ref):
    dis = lax.rsqrt(d_ref[...] + 1.0)
    o_ref[...] = jnp.maximum(dis * (s_ref[...] + g_ref[...]) + b_ref[...], 0.0)


_row_spec = pl.BlockSpec((BR, D), lambda i: (i, 0))
_w_spec = pl.BlockSpec((D, D), lambda i: (0, 0))
_b_spec = pl.BlockSpec((1, D), lambda i: (0, 0))
_out_sds = jax.ShapeDtypeStruct((NPAD, D), jnp.float32)
_grid = (NPAD // BR,)


def _tc_first(x, W0, deg):
    return pl.pallas_call(
        _tc_first_body, grid=_grid,
        in_specs=[_row_spec, _w_spec, _row_spec],
        out_specs=_row_spec, out_shape=_out_sds)(x, W0, deg)


def _tc_mid(S, g, deg, b, W):
    return pl.pallas_call(
        _tc_mid_body, grid=_grid,
        in_specs=[_row_spec, _row_spec, _row_spec, _b_spec, _w_spec],
        out_specs=_row_spec, out_shape=_out_sds)(S, g, deg, b, W)


def _tc_final(S, g, deg, b):
    return pl.pallas_call(
        _tc_final_body, grid=_grid,
        in_specs=[_row_spec, _row_spec, _row_spec, _b_spec],
        out_specs=_row_spec, out_shape=_out_sds)(S, g, deg, b)


def kernel(x, edge_index, W0, b0, W1, b1, W2, b2):
    src = edge_index[0].astype(jnp.int32)
    dst = edge_index[1].astype(jnp.int32)
    padv = jnp.full((EPAD - E,), NPAD, jnp.int32)
    srcp = jnp.concatenate([src, padv]).reshape(NS * KCH, CHUNK)
    dstp = jnp.concatenate([dst, padv]).reshape(NS * KCH, CHUNK)
    x_pad = jnp.pad(x, ((0, NPAD - N), (0, 0)))

    deg = _sc_deg()(dstp)
    g0 = _tc_first(x_pad, W0, deg)
    S0 = _sc_scatter()(g0, srcp, dstp)
    g1 = _tc_mid(S0, g0, deg, b0.reshape(1, D), W1)
    S1 = _sc_scatter()(g1, srcp, dstp)
    g2 = _tc_mid(S1, g1, deg, b1.reshape(1, D), W2)
    S2 = _sc_scatter()(g2, srcp, dstp)
    y = _tc_final(S2, g2, deg, b2.reshape(1, D))
    return y[:N]


# speedup vs baseline: 16.1955x; 1.1072x over previous
import functools

import jax
import jax.numpy as jnp
from jax import lax
from jax.experimental import pallas as pl
from jax.experimental.pallas import tpu as pltpu
from jax.experimental.pallas import tpu_sc as plsc

N = 10000
D = 128
E = 320000
NC = 2
NS = 16
NW = NC * NS
NPAD = 10240
NHALF = NPAD // NC
CHUNK = 128
KCH = 160
EPAD = NS * CHUNK * KCH
RPT = NHALF // NS
BR = 1024
NB = 4
NBS = 4
KCH2 = KCH // 2
IGN = -1


def _remap(sidx, didx, off):
    def row(i, _):
        def col(k, _):
            sl = pl.ds(k * 16, 16)
            d = didx[i, sl] - off
            ok = (d >= 0) & (d < NHALF)
            didx[i, sl] = jnp.where(ok, d, IGN)
            sidx[i, sl] = jnp.where(ok, sidx[i, sl], IGN)
            return 0
        return lax.fori_loop(0, CHUNK // 16, col, 0)
    lax.fori_loop(0, sidx.shape[0], row, 0)


def _zero_acc_stripe(rows, acc, sid):
    z = jnp.zeros((16,), jnp.float32)
    def row(i, _):
        def col(k, _):
            rows[i, pl.ds(k * 16, 16)] = z
            return 0
        return lax.fori_loop(0, D // 16, col, 0)
    lax.fori_loop(0, CHUNK, row, 0)
    for off, n in _stripe_pieces():
        pltpu.sync_copy(rows.at[pl.ds(0, n)],
                        acc.at[pl.ds(sid * RPT + off, n)])


def _stripe_pieces():
    pieces, off = [], 0
    while off < RPT:
        n = min(CHUNK, RPT - off)
        pieces.append((off, n))
        off += n
    return pieces


def _writeback(acc, stage, out_hbm, cid, sid):
    for off, n in _stripe_pieces():
        pltpu.sync_copy(acc.at[pl.ds(sid * RPT + off, n)], stage.at[pl.ds(0, n)])
        pltpu.sync_copy(stage.at[pl.ds(0, n)],
                        out_hbm.at[pl.ds(cid * NHALF + sid * RPT + off, n)])


def _sc_deg_body(dst_hbm, out_hbm, didx, ones_v, acc, s0, s1, s2, s3):
    sems = (s0, s1, s2, s3)
    cid = lax.axis_index("c")
    sid = lax.axis_index("s")

    _zero_acc_stripe(ones_v, acc, sid)
    one = jnp.ones((16,), jnp.float32)
    def fill(i, _):
        def col(k, _):
            ones_v[i, pl.ds(k * 16, 16)] = one
            return 0
        return lax.fori_loop(0, D // 16, col, 0)
    lax.fori_loop(0, CHUNK, fill, 0)
    pltpu.sync_copy(dst_hbm.at[pl.ds(sid * KCH, KCH)], didx)
    _remap(didx, didx, cid * NHALF)
    plsc.subcore_barrier()

    def group(jj, _):
        cps = [pltpu.async_copy(
                   ones_v,
                   acc.at[plsc.Indices(didx.at[jj * NB + b], ignored_value=IGN)],
                   sems[b], add=True)
               for b in range(NB)]
        for cp in cps:
            cp.wait()
        return 0

    lax.fori_loop(0, KCH // NB, group, 0)
    plsc.subcore_barrier()
    _writeback(acc, ones_v, out_hbm, cid, sid)


@functools.lru_cache(maxsize=None)
def _sc_deg():
    return pl.kernel(
        _sc_deg_body,
        out_type=jax.ShapeDtypeStruct((NPAD, D), jnp.float32),
        mesh=plsc.VectorSubcoreMesh(core_axis_name="c", subcore_axis_name="s"),
        scratch_types=[
            pltpu.VMEM((KCH, CHUNK), jnp.int32),
            pltpu.VMEM((CHUNK, D), jnp.float32),
            pltpu.VMEM_SHARED((NHALF, D), jnp.float32),
        ] + [pltpu.SemaphoreType.DMA] * NB,
    )


def _sc_scatter_body(g_hbm, src_hbm, dst_hbm, out_hbm, sidx, didx, rows,
                     acc, g0, g1, g2, g3, t0, t1, t2, t3):
    gsems = (g0, g1, g2, g3)
    ssems = (t0, t1, t2, t3)
    cid = lax.axis_index("c")
    sid = lax.axis_index("s")

    _zero_acc_stripe(rows.at[0], acc, sid)

    def gather(j, b):
        return pltpu.async_copy(
            g_hbm.at[plsc.Indices(sidx.at[j], ignored_value=IGN)],
            rows.at[b], gsems[b])

    def scatter(j, b):
        return pltpu.async_copy(
            rows.at[b],
            acc.at[plsc.Indices(didx.at[j], ignored_value=IGN)],
            ssems[b], add=True)

    def group(jj, _):
        cps = [gather(jj * NBS + b, b) for b in range(NBS)]
        scs = []
        for b in range(NBS):
            cps[b].wait()
            scs.append(scatter(jj * NBS + b, b))
        for sc in scs:
            sc.wait()
        return 0

    for h in range(2):
        pltpu.sync_copy(src_hbm.at[pl.ds(sid * KCH + h * KCH2, KCH2)], sidx)
        pltpu.sync_copy(dst_hbm.at[pl.ds(sid * KCH + h * KCH2, KCH2)], didx)
        _remap(sidx, didx, cid * NHALF)
        lax.fori_loop(0, KCH2 // NBS, group, 0)

    plsc.subcore_barrier()
    _writeback(acc, rows.at[0], out_hbm, cid, sid)


@functools.lru_cache(maxsize=None)
def _sc_scatter():
    return pl.kernel(
        _sc_scatter_body,
        out_type=jax.ShapeDtypeStruct((NPAD, D), jnp.float32),
        mesh=plsc.VectorSubcoreMesh(core_axis_name="c", subcore_axis_name="s"),
        scratch_types=[
            pltpu.VMEM((KCH2, CHUNK), jnp.int32),
            pltpu.VMEM((KCH2, CHUNK), jnp.int32),
            pltpu.VMEM((NBS, CHUNK, D), jnp.float32),
            pltpu.VMEM_SHARED((NHALF, D), jnp.float32),
        ] + [pltpu.SemaphoreType.DMA] * (2 * NBS),
    )


def _tc_first_body(x_ref, w_ref, d_ref, o_ref):
    dis = lax.rsqrt(d_ref[...] + 1.0)
    o_ref[...] = dis * jnp.dot(x_ref[...], w_ref[...],
                               preferred_element_type=jnp.float32)


def _tc_mid_body(s_ref, g_ref, d_ref, b_ref, w_ref, o_ref):
    dis = lax.rsqrt(d_ref[...] + 1.0)
    y = jnp.maximum(dis * (s_ref[...] + g_ref[...]) + b_ref[...], 0.0)
    o_ref[...] = dis * jnp.dot(y, w_ref[...], preferred_element_type=jnp.float32)


def _tc_final_body(s_ref, g_ref, d_ref, b_ref, o_ref):
    dis = lax.rsqrt(d_ref[...] + 1.0)
    o_ref[...] = jnp.maximum(dis * (s_ref[...] + g_ref[...]) + b_ref[...], 0.0)


_row_spec = pl.BlockSpec((BR, D), lambda i: (i, 0))
_w_spec = pl.BlockSpec((D, D), lambda i: (0, 0))
_b_spec = pl.BlockSpec((1, D), lambda i: (0, 0))
_out_sds = jax.ShapeDtypeStruct((NPAD, D), jnp.float32)
_grid = (NPAD // BR,)


def _tc_first(x, W0, deg):
    return pl.pallas_call(
        _tc_first_body, grid=_grid,
        in_specs=[_row_spec, _w_spec, _row_spec],
        out_specs=_row_spec, out_shape=_out_sds)(x, W0, deg)


def _tc_mid(S, g, deg, b, W):
    return pl.pallas_call(
        _tc_mid_body, grid=_grid,
        in_specs=[_row_spec, _row_spec, _row_spec, _b_spec, _w_spec],
        out_specs=_row_spec, out_shape=_out_sds)(S, g, deg, b, W)


def _tc_final(S, g, deg, b):
    return pl.pallas_call(
        _tc_final_body, grid=_grid,
        in_specs=[_row_spec, _row_spec, _row_spec, _b_spec],
        out_specs=_row_spec, out_shape=_out_sds)(S, g, deg, b)


def kernel(x, edge_index, W0, b0, W1, b1, W2, b2):
    src = edge_index[0].astype(jnp.int32)
    dst = edge_index[1].astype(jnp.int32)
    padv = jnp.full((EPAD - E,), NPAD, jnp.int32)
    srcp = jnp.concatenate([src, padv]).reshape(NS * KCH, CHUNK)
    dstp = jnp.concatenate([dst, padv]).reshape(NS * KCH, CHUNK)
    x_pad = jnp.pad(x, ((0, NPAD - N), (0, 0)))

    deg = _sc_deg()(dstp)
    g0 = _tc_first(x_pad, W0, deg)
    S0 = _sc_scatter()(g0, srcp, dstp)
    g1 = _tc_mid(S0, g0, deg, b0.reshape(1, D), W1)
    S1 = _sc_scatter()(g1, srcp, dstp)
    g2 = _tc_mid(S1, g1, deg, b1.reshape(1, D), W2)
    S2 = _sc_scatter()(g2, srcp, dstp)
    y = _tc_final(S2, g2, deg, b2.reshape(1, D))
    return y[:N]
